# Initial kernel scaffold; baseline (speedup 1.0000x reference)
import jax
import jax.numpy as jnp
from jax.experimental import pallas as pl


def kernel(feats, edge_index, graph_ids, W_self1, W_neigh1, b1, W_self2, W_neigh2, b2, w_gate, b_gate, W_p1, b_p1, bn_gamma, bn_beta, bn_mean, bn_var, W_p2, b_p2):
    raise NotImplementedError("write your pallas kernel here")


# speedup vs baseline: 3.7143x; 3.7143x over previous
import functools

import jax
import jax.numpy as jnp
from jax import lax
from jax.experimental import pallas as pl
from jax.experimental.pallas import tpu as pltpu
from jax.experimental.pallas import tpu_sc as plsc

N = 10000
E = 320000
D = 128
H = 64
B = 256
PH = 128

NC = 2
NS = 16
NW = NC * NS
EPW = E // NW
CH = 80
NCHUNK = EPW // CH
NP = 10240
RPS = NP // NS


def _sc_agg_body(with_deg, *refs):
    if with_deg:
        (y_hbm, src_hbm, dst_hbm, z64_hbm, z16_hbm, ones_hbm,
         out_hbm, deg_hbm, src_v, dst_v, rows_v, ones_v, acc_sh, deg_sh,
         sem) = refs
    else:
        (y_hbm, src_hbm, dst_hbm, z64_hbm,
         out_hbm, src_v, dst_v, rows_v, acc_sh,
         sem) = refs
    c = lax.axis_index("c")
    s = lax.axis_index("s")
    wid = s * NC + c
    row0 = s * RPS

    pltpu.sync_copy(z64_hbm.at[pl.ds(row0, RPS), :],
                    acc_sh.at[pl.ds(row0, RPS), :])
    if with_deg:
        pltpu.sync_copy(z16_hbm.at[pl.ds(row0, RPS), :],
                        deg_sh.at[pl.ds(row0, RPS), :])
        pltpu.sync_copy(ones_hbm, ones_v)
    plsc.subcore_barrier()

    base = wid * EPW

    def body(i, carry):
        off = base + i * CH
        pltpu.sync_copy(src_hbm.at[pl.ds(off, CH)], src_v)
        pltpu.sync_copy(dst_hbm.at[pl.ds(off, CH)], dst_v)
        pltpu.async_copy(y_hbm.at[src_v], rows_v, sem).wait()
        pltpu.sync_copy(rows_v, acc_sh.at[dst_v], add=True)
        if with_deg:
            pltpu.sync_copy(ones_v, deg_sh.at[dst_v], add=True)
        return carry

    lax.fori_loop(0, NCHUNK, body, 0)
    plsc.subcore_barrier()

    pltpu.sync_copy(acc_sh.at[pl.ds(row0, RPS), :],
                    out_hbm.at[c, pl.ds(row0, RPS), :])
    if with_deg:
        pltpu.sync_copy(deg_sh.at[pl.ds(row0, RPS), :],
                        deg_hbm.at[c, pl.ds(row0, RPS), :])


def _sc_aggregate(y, src, dst, with_deg):
    mesh = plsc.VectorSubcoreMesh(core_axis_name="c", subcore_axis_name="s",
                                  num_cores=NC)
    yp = jnp.pad(y, ((0, NP - N), (0, 0)))
    z64 = jnp.zeros((NP, H), jnp.float32)
    if with_deg:
        out_type = (jax.ShapeDtypeStruct((NC, NP, H), jnp.float32),
                    jax.ShapeDtypeStruct((NC, NP, 16), jnp.float32))
        scratch = [
            pltpu.VMEM((CH,), jnp.int32),
            pltpu.VMEM((CH,), jnp.int32),
            pltpu.VMEM((CH, H), jnp.float32),
            pltpu.VMEM((CH, 16), jnp.float32),
            pltpu.VMEM_SHARED((NP, H), jnp.float32),
            pltpu.VMEM_SHARED((NP, 16), jnp.float32),
            pltpu.SemaphoreType.DMA,
        ]
        z16 = jnp.zeros((NP, 16), jnp.float32)
        ones = jnp.ones((CH, 16), jnp.float32)
        fn = pl.kernel(functools.partial(_sc_agg_body, True),
                       out_type=out_type, mesh=mesh, scratch_types=scratch,
                       compiler_params=pltpu.CompilerParams(
                           use_tc_tiling_on_sc=False))
        agg, degp = fn(yp, src, dst, z64, z16, ones)
        return agg[:, :N, :], degp[:, :N, :]
    out_type = jax.ShapeDtypeStruct((NC, NP, H), jnp.float32)
    scratch = [
        pltpu.VMEM((CH,), jnp.int32),
        pltpu.VMEM((CH,), jnp.int32),
        pltpu.VMEM((CH, H), jnp.float32),
        pltpu.VMEM_SHARED((NP, H), jnp.float32),
        pltpu.SemaphoreType.DMA,
    ]
    fn = pl.kernel(functools.partial(_sc_agg_body, False),
                   out_type=out_type, mesh=mesh, scratch_types=scratch,
                   compiler_params=pltpu.CompilerParams(
                       use_tc_tiling_on_sc=False))
    return fn(yp, src, dst, z64)[:, :N, :]


RB = 1000


def _k1_body(x_ref, wn_ref, ws_ref, y_ref, s_ref):
    x = x_ref[...]
    y_ref[...] = jnp.dot(x, wn_ref[...], preferred_element_type=jnp.float32)
    s_ref[...] = jnp.dot(x, ws_ref[...], preferred_element_type=jnp.float32)


def _tc_pre(x, w_neigh, w_self):
    grid = (x.shape[0] // RB,)
    bx = pl.BlockSpec((RB, x.shape[1]), lambda i: (i, 0))
    bw = pl.BlockSpec((x.shape[1], H), lambda i: (0, 0))
    bo = pl.BlockSpec((RB, H), lambda i: (i, 0))
    return pl.pallas_call(
        _k1_body,
        grid=grid,
        in_specs=[bx, bw, bw],
        out_specs=[bo, bo],
        out_shape=[jax.ShapeDtypeStruct((x.shape[0], H), jnp.float32),
                   jax.ShapeDtypeStruct((x.shape[0], H), jnp.float32)],
    )(x, w_neigh, w_self)


def _k2_body(s_ref, a0_ref, a1_ref, d0_ref, d1_ref, b_ref, wn_ref, ws_ref,
             y_ref, s2_ref):
    deg = jnp.maximum(d0_ref[:, 0:1] + d1_ref[:, 0:1], 1.0)
    h = jnp.maximum(s_ref[...] + (a0_ref[...] + a1_ref[...]) / deg
                    + b_ref[...], 0.0)
    y_ref[...] = jnp.dot(h, wn_ref[...], preferred_element_type=jnp.float32)
    s2_ref[...] = jnp.dot(h, ws_ref[...], preferred_element_type=jnp.float32)


def _tc_mid(s1, a0, a1, d0, d1, b1, w_neigh2, w_self2):
    grid = (N // RB,)
    bh = pl.BlockSpec((RB, H), lambda i: (i, 0))
    bd = pl.BlockSpec((RB, 16), lambda i: (i, 0))
    bb = pl.BlockSpec((1, H), lambda i: (0, 0))
    bw = pl.BlockSpec((H, H), lambda i: (0, 0))
    return pl.pallas_call(
        _k2_body,
        grid=grid,
        in_specs=[bh, bh, bh, bd, bd, bb, bw, bw],
        out_specs=[bh, bh],
        out_shape=[jax.ShapeDtypeStruct((N, H), jnp.float32),
                   jax.ShapeDtypeStruct((N, H), jnp.float32)],
    )(s1, a0, a1, d0, d1, b1, w_neigh2, w_self2)


def _k3a_body(s_ref, a0_ref, a1_ref, d0_ref, d1_ref, b_ref, wg_ref, bg_ref,
              h_ref, gh_ref):
    deg = jnp.maximum(d0_ref[:, 0:1] + d1_ref[:, 0:1], 1.0)
    h = jnp.maximum(s_ref[...] + (a0_ref[...] + a1_ref[...]) / deg
                    + b_ref[...], 0.0)
    gate = jax.nn.sigmoid(
        jnp.sum(h * wg_ref[...], axis=1, keepdims=True) + bg_ref[...])
    h_ref[...] = h
    gh_ref[...] = gate * h


def _tc_gate(s2, a0, a1, d0, d1, b2, wg_row, bg):
    grid = (N // RB,)
    bh = pl.BlockSpec((RB, H), lambda i: (i, 0))
    bd = pl.BlockSpec((RB, 16), lambda i: (i, 0))
    bb = pl.BlockSpec((1, H), lambda i: (0, 0))
    bg1 = pl.BlockSpec((1, 1), lambda i: (0, 0))
    return pl.pallas_call(
        _k3a_body,
        grid=grid,
        in_specs=[bh, bh, bh, bd, bd, bb, bb, bg1],
        out_specs=[bh, bh],
        out_shape=[jax.ShapeDtypeStruct((N, H), jnp.float32),
                   jax.ShapeDtypeStruct((N, H), jnp.float32)],
    )(s2, a0, a1, d0, d1, b2, wg_row, bg)


SUB = 50


def _k3b_body(h_ref, gh_ref, gid_ref, wp1_ref, bp1_ref, bna_ref, bnc_ref,
              wp2_ref, bp2_ref, out_ref):
    def outer(i, carry):
        wsum, hmax = carry
        gz = gh_ref[pl.ds(i * RB, RB), :]
        gid = gid_ref[pl.ds(i * RB, RB), :]
        segs = lax.broadcasted_iota(jnp.int32, (RB, B), 1)
        mask = (gid == segs)
        wsum = wsum + jax.lax.dot_general(
            mask.astype(jnp.float32), gz, (((0,), (0,)), ((), ())),
            preferred_element_type=jnp.float32)

        def inner(j, hm):
            base = i * RB + j * SUB
            gsub = gid_ref[pl.ds(base, SUB), :]
            hsub = h_ref[pl.ds(base, SUB), :]
            pen = jnp.where(
                gsub == lax.broadcasted_iota(jnp.int32, (SUB, B), 1),
                0.0, -jnp.inf)
            cand = jnp.max(hsub[:, None, :] + pen[:, :, None], axis=0)
            return jnp.maximum(hm, cand)

        hmax = lax.fori_loop(0, RB // SUB, inner, hmax)
        return wsum, hmax

    wsum0 = jnp.zeros((B, H), jnp.float32)
    hmax0 = jnp.full((B, H), -jnp.inf, jnp.float32)
    wsum, hmax = lax.fori_loop(0, N // RB, outer, (wsum0, hmax0))

    g = jnp.concatenate([wsum, hmax], axis=1)
    z = jnp.dot(g, wp1_ref[...], preferred_element_type=jnp.float32)
    z = jnp.maximum(z + bp1_ref[...], 0.0)
    z = z * bna_ref[...] + bnc_ref[...]
    out_ref[...] = jnp.dot(z, wp2_ref[...],
                           preferred_element_type=jnp.float32) + bp2_ref[...]


def _tc_readout(h2, gh, gid2d, wp1, bp1, bn_a, bn_c, wp2, bp2):
    full = lambda a: pl.BlockSpec(a.shape, lambda: tuple(0 for _ in a.shape))
    return pl.pallas_call(
        _k3b_body,
        in_specs=[full(h2), full(gh), full(gid2d), full(wp1), full(bp1),
                  full(bn_a), full(bn_c), full(wp2), full(bp2)],
        out_specs=pl.BlockSpec((B, 1), lambda: (0, 0)),
        out_shape=jax.ShapeDtypeStruct((B, 1), jnp.float32),
    )(h2, gh, gid2d, wp1, bp1, bn_a, bn_c, wp2, bp2)


@jax.jit
def kernel(feats, edge_index, graph_ids, W_self1, W_neigh1, b1, W_self2,
           W_neigh2, b2, w_gate, b_gate, W_p1, b_p1, bn_gamma, bn_beta,
           bn_mean, bn_var, W_p2, b_p2):
    src = edge_index[0]
    dst = edge_index[1]

    y1, s1 = _tc_pre(feats, W_neigh1, W_self1)
    agg1, degp = _sc_aggregate(y1, src, dst, with_deg=True)
    y2, s2 = _tc_mid(s1, agg1[0], agg1[1], degp[0], degp[1],
                     b1.reshape(1, H), W_neigh2, W_self2)
    agg2 = _sc_aggregate(y2, src, dst, with_deg=False)
    h2, gh = _tc_gate(s2, agg2[0], agg2[1], degp[0], degp[1],
                      b2.reshape(1, H), w_gate.reshape(1, H),
                      b_gate.reshape(1, 1))

    bn_a = (bn_gamma * lax.rsqrt(bn_var + 1e-5)).reshape(1, PH)
    bn_c = (bn_beta - bn_mean * bn_gamma * lax.rsqrt(bn_var + 1e-5)
            ).reshape(1, PH)
    out = _tc_readout(h2, gh, graph_ids.reshape(N, 1).astype(jnp.int32),
                      W_p1, b_p1.reshape(1, PH), bn_a, bn_c,
                      W_p2, b_p2.reshape(1, 1))
    return out


# speedup vs baseline: 5.5488x; 1.4939x over previous
import functools

import jax
import jax.numpy as jnp
from jax import lax
from jax.experimental import pallas as pl
from jax.experimental.pallas import tpu as pltpu
from jax.experimental.pallas import tpu_sc as plsc

N = 10000
E = 320000
D = 128
H = 64
B = 256
PH = 128

NC = 2
NS = 16
NW = NC * NS
EPW = E // NW
CH = 80
NCHUNK = EPW // CH
NP = 10240
RPS = NP // NS


def _sc_agg_body(with_deg, *refs):
    if with_deg:
        (y_hbm, src_hbm, dst_hbm, z64_hbm, z16_hbm, ones_hbm,
         out_hbm, deg_hbm, src_t, dst_t, rows0, rows1, ones_v, acc_sh,
         deg_sh, sem0, sem1) = refs
    else:
        (y_hbm, src_hbm, dst_hbm, z64_hbm,
         out_hbm, src_t, dst_t, rows0, rows1, acc_sh,
         sem0, sem1) = refs
    c = lax.axis_index("c")
    s = lax.axis_index("s")
    wid = s * NC + c
    row0 = s * RPS

    pltpu.sync_copy(src_hbm.at[wid], src_t)
    pltpu.sync_copy(dst_hbm.at[wid], dst_t)
    pltpu.sync_copy(z64_hbm.at[pl.ds(row0, RPS), :],
                    acc_sh.at[pl.ds(row0, RPS), :])
    if with_deg:
        pltpu.sync_copy(z16_hbm.at[pl.ds(row0, RPS), :],
                        deg_sh.at[pl.ds(row0, RPS), :])
        pltpu.sync_copy(ones_hbm, ones_v)
    plsc.subcore_barrier()

    def fire(g, buf, sem):
        pltpu.async_copy(y_hbm.at[src_t.at[g]], buf, sem)

    def drain(g, buf, sem):
        pltpu.make_async_copy(y_hbm.at[src_t.at[g]], buf, sem).wait()
        pltpu.sync_copy(buf, acc_sh.at[dst_t.at[g]], add=True)
        if with_deg:
            pltpu.sync_copy(ones_v, deg_sh.at[dst_t.at[g]], add=True)

    fire(0, rows0, sem0)

    def body(i, carry):
        g = 2 * i
        fire(g + 1, rows1, sem1)
        drain(g, rows0, sem0)
        fire(g + 2, rows0, sem0)
        drain(g + 1, rows1, sem1)
        return carry

    lax.fori_loop(0, (NCHUNK - 1) // 2, body, 0)
    drain(NCHUNK - 1, rows0, sem0)
    plsc.subcore_barrier()

    pltpu.sync_copy(acc_sh.at[pl.ds(row0, RPS), :],
                    out_hbm.at[c, pl.ds(row0, RPS), :])
    if with_deg:
        pltpu.sync_copy(deg_sh.at[pl.ds(row0, RPS), :],
                        deg_hbm.at[c, pl.ds(row0, RPS), :])


def _sc_aggregate(y, src, dst, with_deg):
    mesh = plsc.VectorSubcoreMesh(core_axis_name="c", subcore_axis_name="s",
                                  num_cores=NC)
    yp = jnp.pad(y, ((0, NP - N), (0, 0)))
    srcr = src.reshape(NW, NCHUNK, CH)
    dstr = dst.reshape(NW, NCHUNK, CH)
    z64 = jnp.zeros((NP, H), jnp.float32)
    if with_deg:
        out_type = (jax.ShapeDtypeStruct((NC, NP, H), jnp.float32),
                    jax.ShapeDtypeStruct((NC, NP, 16), jnp.float32))
        scratch = [
            pltpu.VMEM((NCHUNK, CH), jnp.int32),
            pltpu.VMEM((NCHUNK, CH), jnp.int32),
            pltpu.VMEM((CH, H), jnp.float32),
            pltpu.VMEM((CH, H), jnp.float32),
            pltpu.VMEM((CH, 16), jnp.float32),
            pltpu.VMEM_SHARED((NP, H), jnp.float32),
            pltpu.VMEM_SHARED((NP, 16), jnp.float32),
            pltpu.SemaphoreType.DMA,
            pltpu.SemaphoreType.DMA,
        ]
        z16 = jnp.zeros((NP, 16), jnp.float32)
        ones = jnp.ones((CH, 16), jnp.float32)
        fn = pl.kernel(functools.partial(_sc_agg_body, True),
                       out_type=out_type, mesh=mesh, scratch_types=scratch,
                       compiler_params=pltpu.CompilerParams(
                           use_tc_tiling_on_sc=False))
        agg, degp = fn(yp, srcr, dstr, z64, z16, ones)
        return agg[:, :N, :], degp[:, :N, :]
    out_type = jax.ShapeDtypeStruct((NC, NP, H), jnp.float32)
    scratch = [
        pltpu.VMEM((NCHUNK, CH), jnp.int32),
        pltpu.VMEM((NCHUNK, CH), jnp.int32),
        pltpu.VMEM((CH, H), jnp.float32),
        pltpu.VMEM((CH, H), jnp.float32),
        pltpu.VMEM_SHARED((NP, H), jnp.float32),
        pltpu.SemaphoreType.DMA,
        pltpu.SemaphoreType.DMA,
    ]
    fn = pl.kernel(functools.partial(_sc_agg_body, False),
                   out_type=out_type, mesh=mesh, scratch_types=scratch,
                   compiler_params=pltpu.CompilerParams(
                       use_tc_tiling_on_sc=False))
    return fn(yp, srcr, dstr, z64)[:, :N, :]


RB = 1000


def _k1_body(x_ref, wn_ref, ws_ref, y_ref, s_ref):
    x = x_ref[...]
    y_ref[...] = jnp.dot(x, wn_ref[...], preferred_element_type=jnp.float32)
    s_ref[...] = jnp.dot(x, ws_ref[...], preferred_element_type=jnp.float32)


def _tc_pre(x, w_neigh, w_self):
    grid = (x.shape[0] // RB,)
    bx = pl.BlockSpec((RB, x.shape[1]), lambda i: (i, 0))
    bw = pl.BlockSpec((x.shape[1], H), lambda i: (0, 0))
    bo = pl.BlockSpec((RB, H), lambda i: (i, 0))
    return pl.pallas_call(
        _k1_body,
        grid=grid,
        in_specs=[bx, bw, bw],
        out_specs=[bo, bo],
        out_shape=[jax.ShapeDtypeStruct((x.shape[0], H), jnp.float32),
                   jax.ShapeDtypeStruct((x.shape[0], H), jnp.float32)],
    )(x, w_neigh, w_self)


def _k2_body(s_ref, a0_ref, a1_ref, d0_ref, d1_ref, b_ref, wn_ref, ws_ref,
             y_ref, s2_ref):
    deg = jnp.maximum(d0_ref[:, 0:1] + d1_ref[:, 0:1], 1.0)
    h = jnp.maximum(s_ref[...] + (a0_ref[...] + a1_ref[...]) / deg
                    + b_ref[...], 0.0)
    y_ref[...] = jnp.dot(h, wn_ref[...], preferred_element_type=jnp.float32)
    s2_ref[...] = jnp.dot(h, ws_ref[...], preferred_element_type=jnp.float32)


def _tc_mid(s1, a0, a1, d0, d1, b1, w_neigh2, w_self2):
    grid = (N // RB,)
    bh = pl.BlockSpec((RB, H), lambda i: (i, 0))
    bd = pl.BlockSpec((RB, 16), lambda i: (i, 0))
    bb = pl.BlockSpec((1, H), lambda i: (0, 0))
    bw = pl.BlockSpec((H, H), lambda i: (0, 0))
    return pl.pallas_call(
        _k2_body,
        grid=grid,
        in_specs=[bh, bh, bh, bd, bd, bb, bw, bw],
        out_specs=[bh, bh],
        out_shape=[jax.ShapeDtypeStruct((N, H), jnp.float32),
                   jax.ShapeDtypeStruct((N, H), jnp.float32)],
    )(s1, a0, a1, d0, d1, b1, w_neigh2, w_self2)


def _k3a_body(s_ref, a0_ref, a1_ref, d0_ref, d1_ref, b_ref, wg_ref, bg_ref,
              h_ref, gh_ref):
    deg = jnp.maximum(d0_ref[:, 0:1] + d1_ref[:, 0:1], 1.0)
    h = jnp.maximum(s_ref[...] + (a0_ref[...] + a1_ref[...]) / deg
                    + b_ref[...], 0.0)
    gate = jax.nn.sigmoid(
        jnp.sum(h * wg_ref[...], axis=1, keepdims=True) + bg_ref[...])
    h_ref[...] = h
    gh_ref[...] = gate * h


def _tc_gate(s2, a0, a1, d0, d1, b2, wg_row, bg):
    grid = (N // RB,)
    bh = pl.BlockSpec((RB, H), lambda i: (i, 0))
    bd = pl.BlockSpec((RB, 16), lambda i: (i, 0))
    bb = pl.BlockSpec((1, H), lambda i: (0, 0))
    bg1 = pl.BlockSpec((1, 1), lambda i: (0, 0))
    return pl.pallas_call(
        _k3a_body,
        grid=grid,
        in_specs=[bh, bh, bh, bd, bd, bb, bb, bg1],
        out_specs=[bh, bh],
        out_shape=[jax.ShapeDtypeStruct((N, H), jnp.float32),
                   jax.ShapeDtypeStruct((N, H), jnp.float32)],
    )(s2, a0, a1, d0, d1, b2, wg_row, bg)


SUB = 50


def _k3b_body(h_ref, gh_ref, gid_ref, wp1_ref, bp1_ref, bna_ref, bnc_ref,
              wp2_ref, bp2_ref, out_ref):
    def outer(i, carry):
        wsum, hmax = carry
        gz = gh_ref[pl.ds(i * RB, RB), :]
        gid = gid_ref[pl.ds(i * RB, RB), :]
        segs = lax.broadcasted_iota(jnp.int32, (RB, B), 1)
        mask = (gid == segs)
        wsum = wsum + jax.lax.dot_general(
            mask.astype(jnp.float32), gz, (((0,), (0,)), ((), ())),
            preferred_element_type=jnp.float32)

        def inner(j, hm):
            base = i * RB + j * SUB
            gsub = gid_ref[pl.ds(base, SUB), :]
            hsub = h_ref[pl.ds(base, SUB), :]
            pen = jnp.where(
                gsub == lax.broadcasted_iota(jnp.int32, (SUB, B), 1),
                0.0, -jnp.inf)
            cand = jnp.max(hsub[:, None, :] + pen[:, :, None], axis=0)
            return jnp.maximum(hm, cand)

        hmax = lax.fori_loop(0, RB // SUB, inner, hmax)
        return wsum, hmax

    wsum0 = jnp.zeros((B, H), jnp.float32)
    hmax0 = jnp.full((B, H), -jnp.inf, jnp.float32)
    wsum, hmax = lax.fori_loop(0, N // RB, outer, (wsum0, hmax0))

    g = jnp.concatenate([wsum, hmax], axis=1)
    z = jnp.dot(g, wp1_ref[...], preferred_element_type=jnp.float32)
    z = jnp.maximum(z + bp1_ref[...], 0.0)
    z = z * bna_ref[...] + bnc_ref[...]
    out_ref[...] = jnp.dot(z, wp2_ref[...],
                           preferred_element_type=jnp.float32) + bp2_ref[...]


def _tc_readout(h2, gh, gid2d, wp1, bp1, bn_a, bn_c, wp2, bp2):
    full = lambda a: pl.BlockSpec(a.shape, lambda: tuple(0 for _ in a.shape))
    return pl.pallas_call(
        _k3b_body,
        in_specs=[full(h2), full(gh), full(gid2d), full(wp1), full(bp1),
                  full(bn_a), full(bn_c), full(wp2), full(bp2)],
        out_specs=pl.BlockSpec((B, 1), lambda: (0, 0)),
        out_shape=jax.ShapeDtypeStruct((B, 1), jnp.float32),
    )(h2, gh, gid2d, wp1, bp1, bn_a, bn_c, wp2, bp2)


@jax.jit
def kernel(feats, edge_index, graph_ids, W_self1, W_neigh1, b1, W_self2,
           W_neigh2, b2, w_gate, b_gate, W_p1, b_p1, bn_gamma, bn_beta,
           bn_mean, bn_var, W_p2, b_p2):
    src = edge_index[0]
    dst = edge_index[1]

    y1, s1 = _tc_pre(feats, W_neigh1, W_self1)
    agg1, degp = _sc_aggregate(y1, src, dst, with_deg=True)
    y2, s2 = _tc_mid(s1, agg1[0], agg1[1], degp[0], degp[1],
                     b1.reshape(1, H), W_neigh2, W_self2)
    agg2 = _sc_aggregate(y2, src, dst, with_deg=False)
    h2, gh = _tc_gate(s2, agg2[0], agg2[1], degp[0], degp[1],
                      b2.reshape(1, H), w_gate.reshape(1, H),
                      b_gate.reshape(1, 1))

    bn_a = (bn_gamma * lax.rsqrt(bn_var + 1e-5)).reshape(1, PH)
    bn_c = (bn_beta - bn_mean * bn_gamma * lax.rsqrt(bn_var + 1e-5)
            ).reshape(1, PH)
    out = _tc_readout(h2, gh, graph_ids.reshape(N, 1).astype(jnp.int32),
                      W_p1, b_p1.reshape(1, PH), bn_a, bn_c,
                      W_p2, b_p2.reshape(1, 1))
    return out
